# baseline (speedup 1.0000x reference)
import functools

import jax
import jax.numpy as jnp
from jax import lax
from jax.experimental import pallas as pl
from jax.experimental.pallas import tpu as pltpu
from jax.experimental.pallas import tpu_sc as plsc

B = 16384
D = 64
OUT = 64
VA = 1000

_info = plsc.get_sparse_core_info()
NC = _info.num_cores
NS = _info.num_subcores
NW = NC * NS
BPW = B // NW
CHUNK = 128
NPH = BPW // CHUNK

_mesh = plsc.VectorSubcoreMesh(core_axis_name="c", subcore_axis_name="s")


@functools.partial(
    pl.kernel,
    mesh=_mesh,
    out_type=jax.ShapeDtypeStruct((B, D), jnp.float32),
    scratch_types=[
        pltpu.VMEM((BPW,), jnp.int32),
        pltpu.VMEM((2, CHUNK, D), jnp.float32),
        pltpu.SemaphoreType.DMA,
        pltpu.SemaphoreType.DMA,
        pltpu.SemaphoreType.DMA,
        pltpu.SemaphoreType.DMA,
    ],
)
def _sc_gather(sidx_hbm, stable_hbm, es_hbm, sidx_v, sbuf,
               gsem0, gsem1, wsem0, wsem1):
    gsems = (gsem0, gsem1)
    wsems = (wsem0, wsem1)
    wid = lax.axis_index("s") * NC + lax.axis_index("c")
    base = wid * BPW
    pltpu.sync_copy(sidx_hbm.at[wid], sidx_v)

    def start_gather(p, s):
        def body(j, _):
            vec = sidx_v[pl.ds(p * CHUNK + j * 16, 16)]
            for k in range(16):
                pltpu.make_async_copy(
                    stable_hbm.at[pl.ds(vec[k], 1)],
                    sbuf.at[s].at[pl.ds(j * 16 + k, 1)], gsems[s]).start()
            return 0
        lax.fori_loop(0, CHUNK // 16, body, 0)

    def wait_gather(s):
        pltpu.make_async_copy(
            stable_hbm.at[pl.ds(0, CHUNK)], sbuf.at[s], gsems[s]).wait()

    def start_write(p, s):
        pltpu.make_async_copy(
            sbuf.at[s], es_hbm.at[pl.ds(base + p * CHUNK, CHUNK)],
            wsems[s]).start()

    def wait_write(s):
        pltpu.make_async_copy(
            sbuf.at[s], es_hbm.at[pl.ds(base, CHUNK)], wsems[s]).wait()

    for p in range(NPH):
        s = p % 2
        if p >= 2:
            wait_write(s)
        start_gather(p, s)
        if p >= 1:
            s2 = (p - 1) % 2
            wait_gather(s2)
            start_write(p - 1, s2)
    s_last = (NPH - 1) % 2
    wait_gather(s_last)
    start_write(NPH - 1, s_last)
    wait_write(0)
    wait_write(1)


BLK = 2048


def _mm_body(es_ref, aid_ref, at_ref, w1_ref, w2_ref, b_ref, o_ref):
    iota = lax.broadcasted_iota(jnp.int32, (BLK, VA), 1)
    oh = (aid_ref[...] == iota).astype(jnp.float32)
    ea = jnp.dot(oh, at_ref[...], preferred_element_type=jnp.float32)
    o_ref[...] = (
        jnp.dot(es_ref[...], w1_ref[...], preferred_element_type=jnp.float32)
        + jnp.dot(ea, w2_ref[...], preferred_element_type=jnp.float32)
        + b_ref[...]
    )


_mm = pl.pallas_call(
    _mm_body,
    grid=(B // BLK,),
    in_specs=[
        pl.BlockSpec((BLK, D), lambda i: (i, 0)),
        pl.BlockSpec((BLK, 1), lambda i: (i, 0)),
        pl.BlockSpec((VA, D), lambda i: (0, 0)),
        pl.BlockSpec((D, OUT), lambda i: (0, 0)),
        pl.BlockSpec((D, OUT), lambda i: (0, 0)),
        pl.BlockSpec((1, OUT), lambda i: (0, 0)),
    ],
    out_specs=pl.BlockSpec((BLK, OUT), lambda i: (i, 0)),
    out_shape=jax.ShapeDtypeStruct((B, OUT), jnp.float32),
)


def kernel(state, action, state_table, action_table, W, b):
    sidx = state.astype(jnp.int32).reshape(NW, BPW)
    es = _sc_gather(sidx, state_table)
    w1 = W[:, :D].T
    w2 = W[:, D:].T
    return _mm(es, action.astype(jnp.int32).reshape(B, 1), action_table,
               w1, w2, b.reshape(1, OUT))


# speedup vs baseline: 1.0058x; 1.0058x over previous
import functools

import jax
import jax.numpy as jnp
from jax import lax
from jax.experimental import pallas as pl
from jax.experimental.pallas import tpu as pltpu
from jax.experimental.pallas import tpu_sc as plsc

B = 16384
D = 64
OUT = 64
VA = 1000

_info = plsc.get_sparse_core_info()
NC = _info.num_cores
NS = _info.num_subcores
NW = NC * NS
BPW = B // NW
CHUNK = 128
NPH = BPW // CHUNK

_mesh = plsc.VectorSubcoreMesh(core_axis_name="c", subcore_axis_name="s")


@functools.partial(
    pl.kernel,
    mesh=_mesh,
    out_type=jax.ShapeDtypeStruct((B, D), jnp.float32),
    scratch_types=[
        pltpu.VMEM((BPW,), jnp.int32),
        pltpu.VMEM((BPW, D), jnp.float32),
        [pltpu.SemaphoreType.DMA] * 8,
        pltpu.SemaphoreType.DMA,
    ],
)
def _sc_gather(sidx_hbm, stable_hbm, es_hbm, sidx_v, sbuf, gsems, wsem):
    wid = lax.axis_index("s") * NC + lax.axis_index("c")
    base = wid * BPW
    pltpu.sync_copy(sidx_hbm.at[wid], sidx_v)

    def body(j, _):
        vec = sidx_v[pl.ds(j * 16, 16)]
        for k in range(16):
            pltpu.make_async_copy(
                stable_hbm.at[pl.ds(vec[k], 1)],
                sbuf.at[pl.ds(j * 16 + k, 1)], gsems[k % 8]).start()
        return 0

    lax.fori_loop(0, BPW // 16, body, 0)
    for i in range(8):
        pltpu.make_async_copy(
            stable_hbm.at[pl.ds(0, BPW // 8)],
            sbuf.at[pl.ds(0, BPW // 8)], gsems[i]).wait()
    pltpu.sync_copy(sbuf, es_hbm.at[pl.ds(base, BPW)])


BLK = 2048


def _mm_body(es_ref, aid_ref, at_ref, w1_ref, w2_ref, b_ref, o_ref):
    iota = lax.broadcasted_iota(jnp.int32, (BLK, VA), 1)
    oh = (aid_ref[...] == iota).astype(jnp.float32)
    ea = jnp.dot(oh, at_ref[...], preferred_element_type=jnp.float32)
    o_ref[...] = (
        jnp.dot(es_ref[...], w1_ref[...], preferred_element_type=jnp.float32)
        + jnp.dot(ea, w2_ref[...], preferred_element_type=jnp.float32)
        + b_ref[...]
    )


_mm = pl.pallas_call(
    _mm_body,
    grid=(B // BLK,),
    in_specs=[
        pl.BlockSpec((BLK, D), lambda i: (i, 0)),
        pl.BlockSpec((BLK, 1), lambda i: (i, 0)),
        pl.BlockSpec((VA, D), lambda i: (0, 0)),
        pl.BlockSpec((D, OUT), lambda i: (0, 0)),
        pl.BlockSpec((D, OUT), lambda i: (0, 0)),
        pl.BlockSpec((1, OUT), lambda i: (0, 0)),
    ],
    out_specs=pl.BlockSpec((BLK, OUT), lambda i: (i, 0)),
    out_shape=jax.ShapeDtypeStruct((B, OUT), jnp.float32),
)


def kernel(state, action, state_table, action_table, W, b):
    sidx = state.astype(jnp.int32).reshape(NW, BPW)
    es = _sc_gather(sidx, state_table)
    w1 = W[:, :D].T
    w2 = W[:, D:].T
    return _mm(es, action.astype(jnp.int32).reshape(B, 1), action_table,
               w1, w2, b.reshape(1, OUT))
